# Initial kernel scaffold; baseline (speedup 1.0000x reference)
import jax
import jax.numpy as jnp
from jax.experimental import pallas as pl


def kernel(x, edge_index, W1, b1, W2, b2):
    raise NotImplementedError("write your pallas kernel here")


# speedup vs baseline: 21.3612x; 21.3612x over previous
import functools

import jax
import jax.numpy as jnp
from jax import lax
from jax.experimental import pallas as pl
from jax.experimental.pallas import tpu as pltpu
from jax.experimental.pallas import tpu_sc as plsc

N = 10000
NPAD = 10240
E = 320000
NC = 2
NS = 16
NW = NC * NS
B = 128
ROWS = E // B
RPT = -(-ROWS // NW)
NPT = NPAD // NS

_MESH = dict(core_axis_name="c", subcore_axis_name="s")
_SC_PARAMS = pltpu.CompilerParams(use_tc_tiling_on_sc=False)


def _deg_call(dst2, ones_b, zeros1):
  @functools.partial(
      pl.kernel,
      out_type=jax.ShapeDtypeStruct((NC * NPAD,), jnp.float32),
      mesh=plsc.VectorSubcoreMesh(**_MESH),
      compiler_params=_SC_PARAMS,
      scratch_types=[
          pltpu.VMEM((1, B), jnp.int32),
          pltpu.VMEM((B,), jnp.float32),
          pltpu.VMEM_SHARED((NPAD,), jnp.float32),
      ],
  )
  def deg_kernel(dst_hbm, ones_hbm, zeros_hbm, out_hbm, didx, ones_v, acc):
    cid = lax.axis_index("c")
    sid = lax.axis_index("s")
    wid = cid * NS + sid
    pltpu.sync_copy(ones_hbm, ones_v)
    pltpu.sync_copy(zeros_hbm.at[pl.ds(sid * NPT, NPT)],
                    acc.at[pl.ds(sid * NPT, NPT)])
    plsc.subcore_barrier()

    def body(i, carry):
      r = wid + i * NW

      @pl.when(r < ROWS)
      def _():
        pltpu.sync_copy(dst_hbm.at[pl.ds(r, 1)], didx)
        pltpu.sync_copy(ones_v, acc.at[didx.at[0]], add=True)

      return carry

    lax.fori_loop(0, RPT, body, 0)
    plsc.subcore_barrier()
    pltpu.sync_copy(acc.at[pl.ds(sid * NPT, NPT)],
                    out_hbm.at[pl.ds(cid * NPAD + sid * NPT, NPT)])

  return deg_kernel(dst2, ones_b, zeros1)


def _make_scatter(D):
  @functools.partial(
      pl.kernel,
      out_type=jax.ShapeDtypeStruct((NC, NPAD, D), jnp.float32),
      mesh=plsc.VectorSubcoreMesh(**_MESH),
      compiler_params=_SC_PARAMS,
      scratch_types=[
          pltpu.VMEM((1, B), jnp.int32),
          pltpu.VMEM((1, B), jnp.int32),
          pltpu.VMEM((B, D), jnp.float32),
          pltpu.VMEM_SHARED((NPAD, D), jnp.float32),
          pltpu.SemaphoreType.DMA,
      ],
  )
  def scatter_kernel(src_hbm, dst_hbm, g_hbm, zeros_hbm, out_hbm,
                     sidx, didx, rows, acc, sem):
    cid = lax.axis_index("c")
    sid = lax.axis_index("s")
    wid = cid * NS + sid
    pltpu.sync_copy(zeros_hbm.at[pl.ds(sid * NPT, NPT)],
                    acc.at[pl.ds(sid * NPT, NPT)])
    plsc.subcore_barrier()

    def body(i, carry):
      r = wid + i * NW

      @pl.when(r < ROWS)
      def _():
        pltpu.sync_copy(src_hbm.at[pl.ds(r, 1)], sidx)
        pltpu.sync_copy(dst_hbm.at[pl.ds(r, 1)], didx)
        pltpu.async_copy(g_hbm.at[sidx.at[0]], rows, sem).wait()
        pltpu.sync_copy(rows, acc.at[didx.at[0]], add=True)

      return carry

    lax.fori_loop(0, RPT, body, 0)
    plsc.subcore_barrier()
    pltpu.sync_copy(acc.at[pl.ds(sid * NPT, NPT)],
                    out_hbm.at[cid, pl.ds(sid * NPT, NPT)])

  return scatter_kernel


_R = 2000


def _tc_head_body(deg_ref, x_ref, w1_ref, g1_ref, dinv_ref):
  deg = jnp.sum(deg_ref[...], axis=1, keepdims=True) + 1.0
  dinv = lax.rsqrt(deg)
  hw = jnp.dot(x_ref[...], w1_ref[...], preferred_element_type=jnp.float32)
  g1_ref[...] = hw * dinv
  dinv_ref[...] = dinv


def _tc_head(degT, x, W1):
  return pl.pallas_call(
      _tc_head_body,
      grid=(N // _R,),
      in_specs=[
          pl.BlockSpec((_R, NC), lambda i: (i, 0)),
          pl.BlockSpec((_R, 128), lambda i: (i, 0)),
          pl.BlockSpec((128, 32), lambda i: (0, 0)),
      ],
      out_specs=[
          pl.BlockSpec((_R, 32), lambda i: (i, 0)),
          pl.BlockSpec((_R, 1), lambda i: (i, 0)),
      ],
      out_shape=[
          jax.ShapeDtypeStruct((N, 32), jnp.float32),
          jax.ShapeDtypeStruct((N, 1), jnp.float32),
      ],
  )(degT, x, W1)


def _tc_mid_body(s_ref, g1_ref, dinv_ref, b1_ref, w2_ref, g2_ref):
  s = s_ref[0] + s_ref[1] + g1_ref[...]
  h = jnp.maximum(s * dinv_ref[...] + b1_ref[...], 0.0)
  g2_ref[...] = jnp.dot(h, w2_ref[...],
                        preferred_element_type=jnp.float32) * dinv_ref[...]


def _tc_mid(s1, g1, dinv, b1, W2):
  return pl.pallas_call(
      _tc_mid_body,
      grid=(N // _R,),
      in_specs=[
          pl.BlockSpec((NC, _R, 32), lambda i: (0, i, 0)),
          pl.BlockSpec((_R, 32), lambda i: (i, 0)),
          pl.BlockSpec((_R, 1), lambda i: (i, 0)),
          pl.BlockSpec((1, 32), lambda i: (0, 0)),
          pl.BlockSpec((32, 16), lambda i: (0, 0)),
      ],
      out_specs=pl.BlockSpec((_R, 16), lambda i: (i, 0)),
      out_shape=jax.ShapeDtypeStruct((N, 16), jnp.float32),
  )(s1, g1, dinv, b1, W2)


def _tc_tail_body(s_ref, g2_ref, dinv_ref, b2_ref, out_ref):
  out_ref[...] = (s_ref[0] + s_ref[1] + g2_ref[...]) * dinv_ref[...] + b2_ref[...]


def _tc_tail(s2, g2, dinv, b2):
  return pl.pallas_call(
      _tc_tail_body,
      grid=(N // _R,),
      in_specs=[
          pl.BlockSpec((NC, _R, 16), lambda i: (0, i, 0)),
          pl.BlockSpec((_R, 16), lambda i: (i, 0)),
          pl.BlockSpec((_R, 1), lambda i: (i, 0)),
          pl.BlockSpec((1, 16), lambda i: (0, 0)),
      ],
      out_specs=pl.BlockSpec((_R, 16), lambda i: (i, 0)),
      out_shape=jax.ShapeDtypeStruct((N, 16), jnp.float32),
  )(s2, g2, dinv, b2)


def kernel(x, edge_index, W1, b1, W2, b2):
  src2 = edge_index[0].reshape(ROWS, B)
  dst2 = edge_index[1].reshape(ROWS, B)
  ones_b = jnp.ones((B,), jnp.float32)
  z1 = jnp.zeros((NPAD,), jnp.float32)
  z32 = jnp.zeros((NPAD, 32), jnp.float32)
  z16 = jnp.zeros((NPAD, 16), jnp.float32)

  degp = _deg_call(dst2, ones_b, z1)
  degT = degp.reshape(NC, NPAD).T
  g1, dinv = _tc_head(degT, x, W1)
  s1 = _make_scatter(32)(src2, dst2, g1, z32)
  g2 = _tc_mid(s1, g1, dinv, b1.reshape(1, 32), W2)
  s2 = _make_scatter(16)(src2, dst2, g2, z16)
  return _tc_tail(s2, g2, dinv, b2.reshape(1, 16))


# speedup vs baseline: 32.5901x; 1.5257x over previous
import functools

import jax
import jax.numpy as jnp
from jax import lax
from jax.experimental import pallas as pl
from jax.experimental.pallas import tpu as pltpu
from jax.experimental.pallas import tpu_sc as plsc

N = 10000
NPAD = 10240
E = 320000
NC = 2
NS = 16
NW = NC * NS
B = 128
RPT = 80
ROWS = NW * RPT
E_PAD = ROWS * B
NBUF = 8
NROUND = RPT // NBUF
NPT = NPAD // NS

_MESH = dict(core_axis_name="c", subcore_axis_name="s")
_SC_PARAMS = pltpu.CompilerParams(use_tc_tiling_on_sc=False)


def _deg_call(dst2, ones_b, zeros1):
  @functools.partial(
      pl.kernel,
      out_type=jax.ShapeDtypeStruct((NC * NPAD,), jnp.float32),
      mesh=plsc.VectorSubcoreMesh(**_MESH),
      compiler_params=_SC_PARAMS,
      scratch_types=[
          pltpu.VMEM((RPT, B), jnp.int32),
          pltpu.VMEM((B,), jnp.float32),
          pltpu.VMEM_SHARED((NPAD,), jnp.float32),
          pltpu.SemaphoreType.DMA,
      ],
  )
  def deg_kernel(dst_hbm, ones_hbm, zeros_hbm, out_hbm, didx, ones_v, acc, sem):
    cid = lax.axis_index("c")
    sid = lax.axis_index("s")
    wid = cid * NS + sid
    pltpu.sync_copy(ones_hbm, ones_v)
    pltpu.sync_copy(dst_hbm.at[pl.ds(wid * RPT, RPT)], didx)
    pltpu.sync_copy(zeros_hbm.at[pl.ds(sid * NPT, NPT)],
                    acc.at[pl.ds(sid * NPT, NPT)])
    plsc.subcore_barrier()

    def body(j, carry):
      i0 = j * NBUF
      descs = [
          pltpu.async_copy(ones_v, acc.at[didx.at[i0 + k]], sem, add=True)
          for k in range(NBUF)
      ]
      for d in descs:
        d.wait()
      return carry

    lax.fori_loop(0, NROUND, body, 0)
    plsc.subcore_barrier()
    pltpu.sync_copy(acc.at[pl.ds(sid * NPT, NPT)],
                    out_hbm.at[pl.ds(cid * NPAD + sid * NPT, NPT)])

  return deg_kernel(dst2, ones_b, zeros1)


def _make_scatter(D):
  @functools.partial(
      pl.kernel,
      out_type=jax.ShapeDtypeStruct((NC, NPAD, D), jnp.float32),
      mesh=plsc.VectorSubcoreMesh(**_MESH),
      compiler_params=_SC_PARAMS,
      scratch_types=[
          pltpu.VMEM((RPT, B), jnp.int32),
          pltpu.VMEM((RPT, B), jnp.int32),
          pltpu.VMEM((NBUF, B, D), jnp.float32),
          pltpu.VMEM_SHARED((NPAD, D), jnp.float32),
          pltpu.SemaphoreType.DMA,
          pltpu.SemaphoreType.DMA,
      ],
  )
  def scatter_kernel(src_hbm, dst_hbm, g_hbm, zeros_hbm, out_hbm,
                     sidx, didx, rows, acc, gsem, ssem):
    cid = lax.axis_index("c")
    sid = lax.axis_index("s")
    wid = cid * NS + sid
    pltpu.sync_copy(src_hbm.at[pl.ds(wid * RPT, RPT)], sidx)
    pltpu.sync_copy(dst_hbm.at[pl.ds(wid * RPT, RPT)], didx)
    pltpu.sync_copy(zeros_hbm.at[pl.ds(sid * NPT, NPT)],
                    acc.at[pl.ds(sid * NPT, NPT)])
    plsc.subcore_barrier()

    def body(j, carry):
      i0 = j * NBUF
      gds = [
          pltpu.async_copy(g_hbm.at[sidx.at[i0 + k]], rows.at[k], gsem)
          for k in range(NBUF)
      ]
      for d in gds:
        d.wait()
      sds = [
          pltpu.async_copy(rows.at[k], acc.at[didx.at[i0 + k]], ssem, add=True)
          for k in range(NBUF)
      ]
      for d in sds:
        d.wait()
      return carry

    lax.fori_loop(0, NROUND, body, 0)
    plsc.subcore_barrier()
    pltpu.sync_copy(acc.at[pl.ds(sid * NPT, NPT)],
                    out_hbm.at[cid, pl.ds(sid * NPT, NPT)])

  return scatter_kernel


_R = 2000


def _tc_head_body(deg_ref, x_ref, w1_ref, g1_ref, dinv_ref):
  deg = jnp.sum(deg_ref[...], axis=1, keepdims=True) + 1.0
  dinv = lax.rsqrt(deg)
  hw = jnp.dot(x_ref[...], w1_ref[...], preferred_element_type=jnp.float32)
  g1_ref[...] = hw * dinv
  dinv_ref[...] = dinv


def _tc_head(degT, x, W1):
  return pl.pallas_call(
      _tc_head_body,
      grid=(N // _R,),
      in_specs=[
          pl.BlockSpec((_R, NC), lambda i: (i, 0)),
          pl.BlockSpec((_R, 128), lambda i: (i, 0)),
          pl.BlockSpec((128, 32), lambda i: (0, 0)),
      ],
      out_specs=[
          pl.BlockSpec((_R, 32), lambda i: (i, 0)),
          pl.BlockSpec((_R, 1), lambda i: (i, 0)),
      ],
      out_shape=[
          jax.ShapeDtypeStruct((NPAD, 32), jnp.float32),
          jax.ShapeDtypeStruct((N, 1), jnp.float32),
      ],
  )(degT, x, W1)


def _tc_mid_body(s_ref, g1_ref, dinv_ref, b1_ref, w2_ref, g2_ref):
  s = s_ref[0] + s_ref[1] + g1_ref[...]
  h = jnp.maximum(s * dinv_ref[...] + b1_ref[...], 0.0)
  g2_ref[...] = jnp.dot(h, w2_ref[...],
                        preferred_element_type=jnp.float32) * dinv_ref[...]


def _tc_mid(s1, g1, dinv, b1, W2):
  return pl.pallas_call(
      _tc_mid_body,
      grid=(N // _R,),
      in_specs=[
          pl.BlockSpec((NC, _R, 32), lambda i: (0, i, 0)),
          pl.BlockSpec((_R, 32), lambda i: (i, 0)),
          pl.BlockSpec((_R, 1), lambda i: (i, 0)),
          pl.BlockSpec((1, 32), lambda i: (0, 0)),
          pl.BlockSpec((32, 16), lambda i: (0, 0)),
      ],
      out_specs=pl.BlockSpec((_R, 16), lambda i: (i, 0)),
      out_shape=jax.ShapeDtypeStruct((NPAD, 16), jnp.float32),
  )(s1, g1, dinv, b1, W2)


def _tc_tail_body(s_ref, g2_ref, dinv_ref, b2_ref, out_ref):
  out_ref[...] = (s_ref[0] + s_ref[1] + g2_ref[...]) * dinv_ref[...] + b2_ref[...]


def _tc_tail(s2, g2, dinv, b2):
  return pl.pallas_call(
      _tc_tail_body,
      grid=(N // _R,),
      in_specs=[
          pl.BlockSpec((NC, _R, 16), lambda i: (0, i, 0)),
          pl.BlockSpec((_R, 16), lambda i: (i, 0)),
          pl.BlockSpec((_R, 1), lambda i: (i, 0)),
          pl.BlockSpec((1, 16), lambda i: (0, 0)),
      ],
      out_specs=pl.BlockSpec((_R, 16), lambda i: (i, 0)),
      out_shape=jax.ShapeDtypeStruct((N, 16), jnp.float32),
  )(s2, g2, dinv, b2)


def kernel(x, edge_index, W1, b1, W2, b2):
  pad = jnp.full((E_PAD - E,), N, jnp.int32)
  src2 = jnp.concatenate([edge_index[0], pad]).reshape(ROWS, B)
  dst2 = jnp.concatenate([edge_index[1], pad]).reshape(ROWS, B)
  ones_b = jnp.ones((B,), jnp.float32)
  z1 = jnp.zeros((NPAD,), jnp.float32)
  z32 = jnp.zeros((NPAD, 32), jnp.float32)
  z16 = jnp.zeros((NPAD, 16), jnp.float32)

  degp = _deg_call(dst2, ones_b, z1)
  degT = degp.reshape(NC, NPAD).T
  g1, dinv = _tc_head(degT, x, W1)
  s1 = _make_scatter(32)(src2, dst2, g1, z32)
  g2 = _tc_mid(s1, g1, dinv, b1.reshape(1, 32), W2)
  s2 = _make_scatter(16)(src2, dst2, g2, z16)
  return _tc_tail(s2, g2, dinv, b2.reshape(1, 16))


# speedup vs baseline: 52.8050x; 1.6203x over previous
import functools

import jax
import jax.numpy as jnp
from jax import lax
from jax.experimental import pallas as pl
from jax.experimental.pallas import tpu as pltpu
from jax.experimental.pallas import tpu_sc as plsc

N = 10000
NPAD = 10240
E = 320000
NC = 2
NS = 16
NW = NC * NS
B = 128
RPT = 80
ROWS = NW * RPT
E_PAD = ROWS * B
NBUF = 8
NROUND = RPT // NBUF
NPT = NPAD // NS

_MESH = dict(core_axis_name="c", subcore_axis_name="s")
_SC_PARAMS = pltpu.CompilerParams(use_tc_tiling_on_sc=False)


def _deg_call(dst2, ones_b, zeros1):
  @functools.partial(
      pl.kernel,
      out_type=jax.ShapeDtypeStruct((NC * NPAD,), jnp.float32),
      mesh=plsc.VectorSubcoreMesh(**_MESH),
      compiler_params=_SC_PARAMS,
      scratch_types=[
          pltpu.VMEM((RPT, B), jnp.int32),
          pltpu.VMEM((B,), jnp.float32),
          pltpu.VMEM_SHARED((NPAD,), jnp.float32),
          pltpu.SemaphoreType.DMA,
      ],
  )
  def deg_kernel(dst_hbm, ones_hbm, zeros_hbm, out_hbm, didx, ones_v, acc, sem):
    cid = lax.axis_index("c")
    sid = lax.axis_index("s")
    wid = cid * NS + sid
    pltpu.sync_copy(ones_hbm, ones_v)
    pltpu.sync_copy(dst_hbm.at[pl.ds(wid * RPT, RPT)], didx)
    pltpu.sync_copy(zeros_hbm.at[pl.ds(sid * NPT, NPT)],
                    acc.at[pl.ds(sid * NPT, NPT)])
    plsc.subcore_barrier()

    def body(j, carry):
      i0 = j * NBUF
      descs = [
          pltpu.async_copy(ones_v, acc.at[didx.at[i0 + k]], sem, add=True)
          for k in range(NBUF)
      ]
      for d in descs:
        d.wait()
      return carry

    lax.fori_loop(0, NROUND, body, 0)
    plsc.subcore_barrier()
    pltpu.sync_copy(acc.at[pl.ds(sid * NPT, NPT)],
                    out_hbm.at[pl.ds(cid * NPAD + sid * NPT, NPT)])

  return deg_kernel(dst2, ones_b, zeros1)


def _make_scatter(D):
  @functools.partial(
      pl.kernel,
      out_type=jax.ShapeDtypeStruct((NC, NPAD, D), jnp.float32),
      mesh=plsc.VectorSubcoreMesh(**_MESH),
      compiler_params=_SC_PARAMS,
      scratch_types=[
          pltpu.VMEM((RPT, B), jnp.int32),
          pltpu.VMEM((RPT, B), jnp.int32),
          pltpu.VMEM((NBUF, B, D), jnp.float32),
          pltpu.VMEM_SHARED((NPAD, D), jnp.float32),
          pltpu.SemaphoreType.DMA,
          pltpu.SemaphoreType.DMA,
      ],
  )
  def scatter_kernel(src_hbm, dst_hbm, g_hbm, zeros_hbm, out_hbm,
                     sidx, didx, rows, acc, gsem, ssem):
    cid = lax.axis_index("c")
    sid = lax.axis_index("s")
    wid = cid * NS + sid
    pltpu.sync_copy(src_hbm.at[pl.ds(wid * RPT, RPT)], sidx)
    pltpu.sync_copy(dst_hbm.at[pl.ds(wid * RPT, RPT)], didx)
    pltpu.sync_copy(zeros_hbm.at[pl.ds(sid * NPT, NPT)],
                    acc.at[pl.ds(sid * NPT, NPT)])
    plsc.subcore_barrier()

    def body(j, carry):
      i0 = j * NBUF
      gds = [
          pltpu.async_copy(g_hbm.at[sidx.at[i0 + k]], rows.at[k], gsem)
          for k in range(NBUF)
      ]
      for d in gds:
        d.wait()
      sds = [
          pltpu.async_copy(rows.at[k], acc.at[didx.at[i0 + k]], ssem, add=True)
          for k in range(NBUF)
      ]
      for d in sds:
        d.wait()
      return carry

    lax.fori_loop(0, NROUND, body, 0)
    plsc.subcore_barrier()
    pltpu.sync_copy(acc.at[pl.ds(sid * NPT, NPT)],
                    out_hbm.at[cid, pl.ds(sid * NPT, NPT)])

  return scatter_kernel


_R = 2000


def _tc_head_body(deg_ref, x_ref, w1_ref, g1_ref, dinv_ref):
  deg = jnp.sum(deg_ref[...], axis=1, keepdims=True) + 1.0
  dinv = lax.rsqrt(deg)
  hw = jnp.dot(x_ref[...], w1_ref[...], preferred_element_type=jnp.float32)
  g1_ref[...] = hw * dinv
  dinv_ref[...] = dinv


def _tc_head(degT, x, W1):
  return pl.pallas_call(
      _tc_head_body,
      grid=(N // _R,),
      in_specs=[
          pl.BlockSpec((_R, NC), lambda i: (i, 0)),
          pl.BlockSpec((_R, 128), lambda i: (i, 0)),
          pl.BlockSpec((128, 32), lambda i: (0, 0)),
      ],
      out_specs=[
          pl.BlockSpec((_R, 32), lambda i: (i, 0)),
          pl.BlockSpec((_R, 1), lambda i: (i, 0)),
      ],
      out_shape=[
          jax.ShapeDtypeStruct((NPAD, 32), jnp.float32),
          jax.ShapeDtypeStruct((N, 1), jnp.float32),
      ],
  )(degT, x, W1)


def _tc_mid_body(s_ref, g1_ref, dinv_ref, b1_ref, w2_ref, g2_ref):
  s = s_ref[0] + s_ref[1] + g1_ref[...]
  h = jnp.maximum(s * dinv_ref[...] + b1_ref[...], 0.0)
  g2_ref[...] = jnp.dot(h, w2_ref[...],
                        preferred_element_type=jnp.float32) * dinv_ref[...]


def _tc_mid(s1, g1, dinv, b1, W2):
  return pl.pallas_call(
      _tc_mid_body,
      grid=(N // _R,),
      in_specs=[
          pl.BlockSpec((NC, _R, 32), lambda i: (0, i, 0)),
          pl.BlockSpec((_R, 32), lambda i: (i, 0)),
          pl.BlockSpec((_R, 1), lambda i: (i, 0)),
          pl.BlockSpec((1, 32), lambda i: (0, 0)),
          pl.BlockSpec((32, 16), lambda i: (0, 0)),
      ],
      out_specs=pl.BlockSpec((_R, 16), lambda i: (i, 0)),
      out_shape=jax.ShapeDtypeStruct((NPAD, 16), jnp.float32),
  )(s1, g1, dinv, b1, W2)


def _tc_tail_body(s_ref, g2_ref, dinv_ref, b2_ref, out_ref):
  out_ref[...] = (s_ref[0] + s_ref[1] + g2_ref[...]) * dinv_ref[...] + b2_ref[...]


def _tc_tail(s2, g2, dinv, b2):
  return pl.pallas_call(
      _tc_tail_body,
      grid=(N // _R,),
      in_specs=[
          pl.BlockSpec((NC, _R, 16), lambda i: (0, i, 0)),
          pl.BlockSpec((_R, 16), lambda i: (i, 0)),
          pl.BlockSpec((_R, 1), lambda i: (i, 0)),
          pl.BlockSpec((1, 16), lambda i: (0, 0)),
      ],
      out_specs=pl.BlockSpec((_R, 16), lambda i: (i, 0)),
      out_shape=jax.ShapeDtypeStruct((N, 16), jnp.float32),
  )(s2, g2, dinv, b2)


def kernel(x, edge_index, W1, b1, W2, b2):
  pad = N + jnp.arange(E_PAD - E, dtype=jnp.int32) % (NPAD - N)
  src2 = jnp.concatenate([edge_index[0], pad]).reshape(ROWS, B)
  dst2 = jnp.concatenate([edge_index[1], pad]).reshape(ROWS, B)
  ones_b = jnp.ones((B,), jnp.float32)
  z1 = jnp.zeros((NPAD,), jnp.float32)
  z32 = jnp.zeros((NPAD, 32), jnp.float32)
  z16 = jnp.zeros((NPAD, 16), jnp.float32)

  degp = _deg_call(dst2, ones_b, z1)
  degT = degp.reshape(NC, NPAD).T
  g1, dinv = _tc_head(degT, x, W1)
  s1 = _make_scatter(32)(src2, dst2, g1, z32)
  g2 = _tc_mid(s1, g1, dinv, b1.reshape(1, 32), W2)
  s2 = _make_scatter(16)(src2, dst2, g2, z16)
  return _tc_tail(s2, g2, dinv, b2.reshape(1, 16))
